# baseline (speedup 1.0000x reference)
import jax
import jax.numpy as jnp
from jax.experimental import pallas as pl
from jax.experimental.pallas import tpu as pltpu

_CHUNK_FRACS = (1, 7)


def _make_body(batch, seq, d, bounds):
    n_chunks = len(bounds) - 1

    def body(w_hbm, out_hbm, w_vmem, in_sems, out_sems):
        in_cps = []
        for c in range(n_chunks):
            sl = pl.ds(bounds[c], bounds[c + 1] - bounds[c])
            cp = pltpu.make_async_copy(
                w_hbm.at[sl, :], w_vmem.at[sl, :], in_sems.at[c])
            cp.start()
            in_cps.append(cp)
        out_cps = []
        for c in range(n_chunks):
            in_cps[c].wait()
            sl = pl.ds(bounds[c], bounds[c + 1] - bounds[c])
            for b in range(batch):
                cp = pltpu.make_async_copy(
                    w_vmem.at[sl, :], out_hbm.at[b, sl, :], out_sems.at[b, c])
                cp.start()
                out_cps.append(cp)
        for cp in out_cps:
            cp.wait()

    return body


def kernel(tokens, W_pos):
    batch, seq = tokens.shape
    d = W_pos.shape[-1]
    total = sum(_CHUNK_FRACS)
    bounds = [0]
    for f in _CHUNK_FRACS:
        bounds.append(bounds[-1] + seq * f // total)
    bounds[-1] = seq
    return pl.pallas_call(
        _make_body(batch, seq, d, bounds),
        in_specs=[pl.BlockSpec(memory_space=pltpu.MemorySpace.HBM)],
        out_specs=pl.BlockSpec(memory_space=pltpu.MemorySpace.HBM),
        out_shape=jax.ShapeDtypeStruct((batch, seq, d), W_pos.dtype),
        scratch_shapes=[
            pltpu.VMEM((seq, d), W_pos.dtype),
            pltpu.SemaphoreType.DMA((len(_CHUNK_FRACS),)),
            pltpu.SemaphoreType.DMA((batch, len(_CHUNK_FRACS))),
        ],
    )(W_pos[:seq])


# speedup vs baseline: 1.0333x; 1.0333x over previous
import jax
import jax.numpy as jnp
from jax.experimental import pallas as pl
from jax.experimental.pallas import tpu as pltpu

_CHUNK_FRACS = (1, 2, 13)


def _make_body(batch, seq, d, bounds):
    n_chunks = len(bounds) - 1

    def body(w_hbm, out_hbm, w_vmem, in_sems, out_sems):
        in_cps = []
        for c in range(n_chunks):
            sl = pl.ds(bounds[c], bounds[c + 1] - bounds[c])
            cp = pltpu.make_async_copy(
                w_hbm.at[sl, :], w_vmem.at[sl, :], in_sems.at[c])
            cp.start()
            in_cps.append(cp)
        out_cps = []
        for c in range(n_chunks):
            in_cps[c].wait()
            sl = pl.ds(bounds[c], bounds[c + 1] - bounds[c])
            for b in range(batch):
                cp = pltpu.make_async_copy(
                    w_vmem.at[sl, :], out_hbm.at[b, sl, :], out_sems.at[b, c])
                cp.start()
                out_cps.append(cp)
        for cp in out_cps:
            cp.wait()

    return body


def kernel(tokens, W_pos):
    batch, seq = tokens.shape
    d = W_pos.shape[-1]
    total = sum(_CHUNK_FRACS)
    bounds = [0]
    for f in _CHUNK_FRACS:
        bounds.append(bounds[-1] + seq * f // total)
    bounds[-1] = seq
    return pl.pallas_call(
        _make_body(batch, seq, d, bounds),
        in_specs=[pl.BlockSpec(memory_space=pltpu.MemorySpace.HBM)],
        out_specs=pl.BlockSpec(memory_space=pltpu.MemorySpace.HBM),
        out_shape=jax.ShapeDtypeStruct((batch, seq, d), W_pos.dtype),
        scratch_shapes=[
            pltpu.VMEM((seq, d), W_pos.dtype),
            pltpu.SemaphoreType.DMA((len(_CHUNK_FRACS),)),
            pltpu.SemaphoreType.DMA((batch, len(_CHUNK_FRACS))),
        ],
    )(W_pos[:seq])
